# Initial kernel scaffold; baseline (speedup 1.0000x reference)
import jax
import jax.numpy as jnp
from jax.experimental import pallas as pl


def kernel(features, adj, batch, labels, temperature, W_self0, W_neigh0, W_att0, W_self1, W_neigh1, W_att1, W_pred, b_pred):
    raise NotImplementedError("write your pallas kernel here")


# speedup vs baseline: 2.0905x; 2.0905x over previous
import functools

import jax
import jax.numpy as jnp
from jax import lax
from jax.experimental import pallas as pl
from jax.experimental.pallas import tpu as pltpu
from jax.experimental.pallas import tpu_sc as plsc

N_NODES_ = 100000
D_ = 128
MAX_DEG_ = 32
B_ = 1024
NS0_ = 25
NS1_ = 10
NCLS_ = 64
WD_ = 1e-4

_NW = 32


def _chunk_for(b_per_w):
    ch = 8
    for c in range(8, 129, 8):
        if b_per_w % c == 0:
            ch = c
    return ch


def _make_sc_gather(n_idx, specs):
    ntab = len(specs)
    b_per_w = n_idx // _NW
    ch = _chunk_for(b_per_w)
    n_iter = b_per_w // ch
    mesh = plsc.VectorSubcoreMesh(core_axis_name="c", subcore_axis_name="s")
    out_type = [jax.ShapeDtypeStruct((n_idx, d), dt) for d, dt in specs]
    scratch = [pltpu.VMEM((ch,), jnp.int32)]
    scratch += [pltpu.VMEM((ch, d), dt) for d, dt in specs]
    scratch += [pltpu.SemaphoreType.DMA for _ in specs]

    @functools.partial(
        pl.kernel, mesh=mesh, out_type=out_type, scratch_types=scratch,
        compiler_params=pltpu.CompilerParams(use_tc_tiling_on_sc=False))
    def gather(*refs):
        tables = refs[:ntab]
        idx_hbm = refs[ntab]
        outs = refs[ntab + 1:2 * ntab + 1]
        idx_v = refs[2 * ntab + 1]
        rows = refs[2 * ntab + 2:3 * ntab + 2]
        sems = refs[3 * ntab + 2:]
        wid = lax.axis_index("s") * 2 + lax.axis_index("c")
        base = wid * b_per_w

        def body(i, carry):
            off = base + i * ch
            pltpu.sync_copy(idx_hbm.at[pl.ds(off, ch)], idx_v)
            copies = [pltpu.async_copy(t.at[idx_v], r, s)
                      for t, r, s in zip(tables, rows, sems)]
            for cp in copies:
                cp.wait()
            for r, o in zip(rows, outs):
                pltpu.sync_copy(r, o.at[pl.ds(off, ch)])
            return carry

        if n_iter == 1:
            body(0, 0)
        else:
            lax.fori_loop(0, n_iter, body, 0)

    return gather


_gather_hop0 = _make_sc_gather(B_, [(MAX_DEG_, jnp.int32), (D_, jnp.float32)])
_gather_hop1 = _make_sc_gather(B_ * NS1_,
                               [(MAX_DEG_, jnp.int32), (D_, jnp.float32)])
_gather_hop2 = _make_sc_gather(B_ * NS1_ * NS0_, [(D_, jnp.float32)])


def _hop1_body(h1_ref, h2_ref, wa_ref, ws_ref, wn_ref, temp_ref, out_ref):
    h1 = h1_ref[...]
    h2 = h2_ref[...]
    q = jnp.dot(h1, wa_ref[...], preferred_element_type=jnp.float32,
                 precision=lax.Precision.HIGHEST)
    scores = jnp.sum(q[:, None, :] * h2, axis=-1) / temp_ref[0]
    m = jnp.max(scores, axis=-1, keepdims=True)
    e = jnp.exp(scores - m)
    att = e / jnp.sum(e, axis=-1, keepdims=True)
    agg = jnp.sum(att[:, :, None] * h2, axis=1)
    fs = jnp.dot(h1, ws_ref[...], preferred_element_type=jnp.float32,
                 precision=lax.Precision.HIGHEST)
    fn = jnp.dot(agg, wn_ref[...], preferred_element_type=jnp.float32,
                 precision=lax.Precision.HIGHEST)
    out_ref[...] = jnp.maximum(jnp.concatenate([fs, fn], axis=-1), 0.0)


_T1 = 1024


def _hop1_call(h1, h2r, w_att0, w_self0, w_neigh0, temp):
    n = B_ * NS1_
    return pl.pallas_call(
        _hop1_body,
        grid=(n // _T1,),
        in_specs=[
            pl.BlockSpec((_T1, D_), lambda i: (i, 0)),
            pl.BlockSpec((_T1, NS0_, D_), lambda i: (i, 0, 0)),
            pl.BlockSpec((D_, D_), lambda i: (0, 0)),
            pl.BlockSpec((D_, D_), lambda i: (0, 0)),
            pl.BlockSpec((D_, D_), lambda i: (0, 0)),
            pl.BlockSpec(memory_space=pltpu.SMEM),
        ],
        out_specs=pl.BlockSpec((_T1, 2 * D_), lambda i: (i, 0)),
        out_shape=jax.ShapeDtypeStruct((n, 2 * D_), jnp.float32),
    )(h1, h2r, w_att0, w_self0, w_neigh0, temp)


def _attn(self_vecs, neigh, w_att, temp):
    q = jnp.dot(self_vecs, w_att, preferred_element_type=jnp.float32,
                 precision=lax.Precision.HIGHEST)
    scores = jnp.sum(q[:, None, :] * neigh, axis=-1) / temp
    m = jnp.max(scores, axis=-1, keepdims=True)
    e = jnp.exp(scores - m)
    att = e / jnp.sum(e, axis=-1, keepdims=True)
    return jnp.sum(att[:, :, None] * neigh, axis=1)


def _argmax_rows(x):
    ids = lax.broadcasted_iota(jnp.int32, x.shape, 1)
    m = jnp.max(x, axis=1, keepdims=True)
    return jnp.min(jnp.where(x == m, ids, x.shape[1]), axis=1)


def _head_body(h0_ref, h1_ref, hh1_ref, labels_ref, temp_ref,
               ws0_ref, wn0_ref, wa0_ref, ws1_ref, wn1_ref, wa1_ref,
               wp_ref, bp_ref, preds_ref, loss_ref, acc_ref):
    temp = temp_ref[0]
    h0 = h0_ref[...]
    n0 = h1_ref[...]
    agg0 = _attn(h0, n0, wa0_ref[...], temp)
    fs0 = jnp.dot(h0, ws0_ref[...], preferred_element_type=jnp.float32,
                 precision=lax.Precision.HIGHEST)
    fn0 = jnp.dot(agg0, wn0_ref[...], preferred_element_type=jnp.float32,
                 precision=lax.Precision.HIGHEST)
    hid0 = jnp.maximum(jnp.concatenate([fs0, fn0], axis=-1), 0.0)
    n1 = hh1_ref[...]
    agg1 = _attn(hid0, n1, wa1_ref[...], temp)
    fs1 = jnp.dot(hid0, ws1_ref[...], preferred_element_type=jnp.float32,
                 precision=lax.Precision.HIGHEST)
    fn1 = jnp.dot(agg1, wn1_ref[...], preferred_element_type=jnp.float32,
                 precision=lax.Precision.HIGHEST)
    h = jnp.concatenate([fs1, fn1], axis=-1)
    nrm = jnp.sqrt(jnp.sum(h * h, axis=1, keepdims=True)) + 1e-12
    out1 = h / nrm
    logits = jnp.dot(out1, wp_ref[...],
                     preferred_element_type=jnp.float32,
                 precision=lax.Precision.HIGHEST) + bp_ref[...]
    lm = jnp.max(logits, axis=-1, keepdims=True)
    ls = logits - lm
    lse = jnp.log(jnp.sum(jnp.exp(ls), axis=-1, keepdims=True))
    logp = ls - lse
    labels = labels_ref[...]
    cross = jnp.mean(-jnp.sum(labels * logp, axis=-1))
    l2 = (jnp.sum(ws0_ref[...] ** 2) + jnp.sum(wn0_ref[...] ** 2)
          + jnp.sum(wa0_ref[...] ** 2) + jnp.sum(ws1_ref[...] ** 2)
          + jnp.sum(wn1_ref[...] ** 2) + jnp.sum(wa1_ref[...] ** 2)
          + jnp.sum(wp_ref[...] ** 2) + jnp.sum(bp_ref[...] ** 2))
    loss = cross + WD_ * 0.5 * l2
    preds = jnp.exp(logp)
    preds_ref[...] = preds
    loss_ref[...] = jnp.reshape(loss, (1, 1))
    acc = jnp.mean((_argmax_rows(preds) == _argmax_rows(labels))
                   .astype(jnp.float32))
    acc_ref[...] = jnp.reshape(acc, (1, 1))


def _head_call(h0, h1r, hh1r, labels, temp, ws0, wn0, wa0, ws1, wn1, wa1,
               wp, bp):
    vmem = lambda: pl.BlockSpec(memory_space=pltpu.ANY)
    return pl.pallas_call(
        _head_body,
        in_specs=[
            pl.BlockSpec((B_, D_), lambda: (0, 0)),
            pl.BlockSpec((B_, NS1_, D_), lambda: (0, 0, 0)),
            pl.BlockSpec((B_, NS1_, 2 * D_), lambda: (0, 0, 0)),
            pl.BlockSpec((B_, NCLS_), lambda: (0, 0)),
            pl.BlockSpec(memory_space=pltpu.SMEM),
            pl.BlockSpec((D_, D_), lambda: (0, 0)),
            pl.BlockSpec((D_, D_), lambda: (0, 0)),
            pl.BlockSpec((D_, D_), lambda: (0, 0)),
            pl.BlockSpec((2 * D_, D_), lambda: (0, 0)),
            pl.BlockSpec((2 * D_, D_), lambda: (0, 0)),
            pl.BlockSpec((2 * D_, 2 * D_), lambda: (0, 0)),
            pl.BlockSpec((2 * D_, NCLS_), lambda: (0, 0)),
            pl.BlockSpec((1, NCLS_), lambda: (0, 0)),
        ],
        out_specs=[
            pl.BlockSpec((B_, NCLS_), lambda: (0, 0)),
            pl.BlockSpec((1, 1), lambda: (0, 0)),
            pl.BlockSpec((1, 1), lambda: (0, 0)),
        ],
        out_shape=[
            jax.ShapeDtypeStruct((B_, NCLS_), jnp.float32),
            jax.ShapeDtypeStruct((1, 1), jnp.float32),
            jax.ShapeDtypeStruct((1, 1), jnp.float32),
        ],
    )(h0, h1r, hh1r, labels, temp, ws0, wn0, wa0, ws1, wn1, wa1, wp, bp)


def kernel(features, adj, batch, labels, temperature, W_self0, W_neigh0,
           W_att0, W_self1, W_neigh1, W_att1, W_pred, b_pred):
    adj0, h0 = _gather_hop0(adj, features, batch)
    s1 = adj0[:, :NS1_].reshape(-1)
    adj1, h1 = _gather_hop1(adj, features, s1)
    s2 = adj1[:, :NS0_].reshape(-1)
    (h2,) = _gather_hop2(features, s2)
    temp = temperature.reshape(1)
    hh1 = _hop1_call(h1, h2.reshape(B_ * NS1_, NS0_, D_),
                     W_att0, W_self0, W_neigh0, temp)
    preds, loss, acc = _head_call(
        h0, h1.reshape(B_, NS1_, D_), hh1.reshape(B_, NS1_, 2 * D_),
        labels, temp, W_self0, W_neigh0, W_att0, W_self1, W_neigh1, W_att1,
        W_pred, b_pred.reshape(1, NCLS_))
    return preds, jnp.reshape(loss, ()), jnp.reshape(acc, ())


# speedup vs baseline: 5.0567x; 2.4189x over previous
import functools

import jax
import jax.numpy as jnp
from jax import lax
from jax.experimental import pallas as pl
from jax.experimental.pallas import tpu as pltpu
from jax.experimental.pallas import tpu_sc as plsc

N_NODES_ = 100000
D_ = 128
MAX_DEG_ = 32
B_ = 1024
NS0_ = 25
NS1_ = 10
NCLS_ = 64
WD_ = 1e-4

_NW = 32
_HP = lax.Precision.HIGHEST


def _chunk_for(b_per_w):
    ch = 8
    for c in range(8, 129, 8):
        if b_per_w % c == 0:
            ch = c
    return ch


def _make_sc_gather(n_idx, specs):
    ntab = len(specs)
    b_per_w = n_idx // _NW
    ch = _chunk_for(b_per_w)
    n_iter = b_per_w // ch
    mesh = plsc.VectorSubcoreMesh(core_axis_name="c", subcore_axis_name="s", num_cores=2, num_subcores=16)
    out_type = [jax.ShapeDtypeStruct((n_idx, d), dt) for d, dt in specs]
    scratch = [pltpu.VMEM((ch,), jnp.int32)]
    scratch += [pltpu.VMEM((ch, d), dt) for d, dt in specs]
    scratch += [pltpu.SemaphoreType.DMA for _ in specs]

    @functools.partial(
        pl.kernel, mesh=mesh, out_type=out_type, scratch_types=scratch,
        compiler_params=pltpu.CompilerParams(use_tc_tiling_on_sc=False))
    def gather(*refs):
        tables = refs[:ntab]
        idx_hbm = refs[ntab]
        outs = refs[ntab + 1:2 * ntab + 1]
        idx_v = refs[2 * ntab + 1]
        rows = refs[2 * ntab + 2:3 * ntab + 2]
        sems = refs[3 * ntab + 2:]
        wid = lax.axis_index("s") * 2 + lax.axis_index("c")
        base = wid * b_per_w

        def body(i, carry):
            off = base + i * ch
            pltpu.sync_copy(idx_hbm.at[pl.ds(off, ch)], idx_v)
            copies = [pltpu.async_copy(t.at[idx_v], r, s)
                      for t, r, s in zip(tables, rows, sems)]
            for cp in copies:
                cp.wait()
            for r, o in zip(rows, outs):
                pltpu.sync_copy(r, o.at[pl.ds(off, ch)])
            return carry

        if n_iter == 1:
            body(0, 0)
        else:
            lax.fori_loop(0, n_iter, body, 0)

    return gather


_gather_hop0 = _make_sc_gather(B_, [(MAX_DEG_, jnp.int32), (D_, jnp.float32)])
_gather_hop1 = _make_sc_gather(B_ * NS1_,
                               [(MAX_DEG_, jnp.int32), (D_, jnp.float32)])


_NT = B_ * NS1_
_TPW = _NT // _NW
_CT = 8
_CI = _CT * NS0_
_NCH = _TPW // _CT


def _make_fused_agg():
    mesh = plsc.VectorSubcoreMesh(core_axis_name="c", subcore_axis_name="s", num_cores=2, num_subcores=16)

    @functools.partial(
        pl.kernel, mesh=mesh,
        out_type=jax.ShapeDtypeStruct((_NT, D_), jnp.float32),
        scratch_types=[
            pltpu.VMEM((_TPW * NS0_,), jnp.int32),
            pltpu.VMEM((_TPW, D_), jnp.float32),
            pltpu.VMEM((2, _CI, D_), jnp.float32),
            pltpu.VMEM((32,), jnp.float32),
            pltpu.VMEM((2, _CT, D_), jnp.float32),
            pltpu.SemaphoreType.DMA,
            pltpu.SemaphoreType.DMA,
            pltpu.SemaphoreType.DMA,
            pltpu.SemaphoreType.DMA,
        ],
        compiler_params=pltpu.CompilerParams(use_tc_tiling_on_sc=False,
                                             needs_layout_passes=False))
    def fused(feat_hbm, s2_hbm, q_hbm, out_hbm,
              idx_a, q_a, rows_v, sc_v, out_v, sg0, sg1, so0, so1):
        wid = lax.axis_index("s") * 2 + lax.axis_index("c")
        base = wid * _TPW
        lanes = lax.iota(jnp.int32, 16)
        sg = (sg0, sg1)
        so = (so0, so1)

        pltpu.sync_copy(s2_hbm.at[pl.ds(base * NS0_, _TPW * NS0_)], idx_a)
        pltpu.sync_copy(q_hbm.at[pl.ds(base, _TPW)], q_a)

        def fire(c, b):
            o = c * _CI
            pltpu.async_copy(feat_hbm.at[idx_a.at[pl.ds(o, 104)]],
                             rows_v.at[b, pl.ds(0, 104)], sg[b])
            pltpu.async_copy(feat_hbm.at[idx_a.at[pl.ds(o + 104, 96)]],
                             rows_v.at[b, pl.ds(104, 96)], sg[b])

        def drain_gather(c, b):
            o = c * _CI
            pltpu.make_async_copy(feat_hbm.at[idx_a.at[pl.ds(o, 104)]],
                                  rows_v.at[b, pl.ds(0, 104)], sg[b]).wait()
            pltpu.make_async_copy(feat_hbm.at[idx_a.at[pl.ds(o + 104, 96)]],
                                  rows_v.at[b, pl.ds(104, 96)], sg[b]).wait()

        def out_start(c, b):
            pltpu.async_copy(out_v.at[b], out_hbm.at[pl.ds(base + c * _CT,
                                                           _CT)], so[b])

        def out_drain(c, b):
            pltpu.make_async_copy(out_v.at[b],
                                  out_hbm.at[pl.ds(base + c * _CT, _CT)],
                                  so[b]).wait()

        def compute(c, b):
            def target_body(j, carry2):
                rbase = j * NS0_
                tl = c * _CT + j
                qv = [q_a[tl, pl.ds(k * 16, 16)] for k in range(8)]

                def dot_body(k, carry3):
                    v0, v1 = carry3
                    r = rbase + k
                    acc = rows_v[b, r, pl.ds(0, 16)] * qv[0]
                    for m_ in range(1, 8):
                        acc = acc + rows_v[b, r, pl.ds(m_ * 16, 16)] * qv[m_]
                    s = jnp.sum(acc, axis=0)
                    v0 = jnp.where(lanes == k, s, v0)
                    v1 = jnp.where(lanes == k - 16, s, v1)
                    return (v0, v1)

                neg = jnp.full((16,), -3e38, jnp.float32)
                s0, s1 = lax.fori_loop(0, NS0_, dot_body, (neg, neg),
                                       unroll=5)
                m = jnp.max(jnp.maximum(s0, s1), axis=0)
                e0 = jnp.exp(s0 - m)
                e1 = jnp.exp(s1 - m)
                tot = jnp.sum(e0 + e1, axis=0)
                sc_v[pl.ds(0, 16)] = e0 / tot
                sc_v[pl.ds(16, 16)] = e1 / tot

                def wsum_body(k, accs):
                    r = rbase + k
                    w = plsc.load_gather(
                        sc_v, [jnp.full((16,), k, jnp.int32)])
                    return tuple(
                        accs[m_] + rows_v[b, r, pl.ds(m_ * 16, 16)] * w
                        for m_ in range(8))

                zero = jnp.zeros((16,), jnp.float32)
                accs = lax.fori_loop(0, NS0_, wsum_body, (zero,) * 8,
                                     unroll=5)
                for m_ in range(8):
                    out_v[b, j, pl.ds(m_ * 16, 16)] = accs[m_]
                return carry2

            lax.fori_loop(0, _CT, target_body, 0)

        fire(0, 0)

        def loop_body(i, carry):
            c0 = 2 * i
            c1 = 2 * i + 1
            fire(c1, 1)
            drain_gather(c0, 0)

            @pl.when(i >= 1)
            def _():
                out_drain(c0 - 2, 0)

            compute(c0, 0)
            out_start(c0, 0)

            @pl.when(i < (_NCH // 2) - 1)
            def _():
                fire(c1 + 1, 0)

            drain_gather(c1, 1)

            @pl.when(i >= 1)
            def _():
                out_drain(c1 - 2, 1)

            compute(c1, 1)
            out_start(c1, 1)
            return carry

        lax.fori_loop(0, _NCH // 2, loop_body, 0)
        out_drain(_NCH - 2, 0)
        out_drain(_NCH - 1, 1)

    return fused


_fused_agg = _make_fused_agg()


def _q_body(h1_ref, wa_ref, temp_ref, out_ref):
    q = jnp.dot(h1_ref[...], wa_ref[...], preferred_element_type=jnp.float32,
                precision=_HP)
    out_ref[...] = q / temp_ref[0]


def _q_call(h1, w_att0, temp):
    n = B_ * NS1_
    return pl.pallas_call(
        _q_body,
        in_specs=[
            pl.BlockSpec((n, D_), lambda: (0, 0)),
            pl.BlockSpec((D_, D_), lambda: (0, 0)),
            pl.BlockSpec(memory_space=pltpu.SMEM),
        ],
        out_specs=pl.BlockSpec((n, D_), lambda: (0, 0)),
        out_shape=jax.ShapeDtypeStruct((n, D_), jnp.float32),
    )(h1, w_att0, temp)


def _hh_body(h1_ref, agg_ref, ws_ref, wn_ref, out_ref):
    fs = jnp.dot(h1_ref[...], ws_ref[...], preferred_element_type=jnp.float32,
                 precision=_HP)
    fn = jnp.dot(agg_ref[...], wn_ref[...],
                 preferred_element_type=jnp.float32, precision=_HP)
    out_ref[...] = jnp.maximum(jnp.concatenate([fs, fn], axis=-1), 0.0)


def _hh_call(h1, agg1, w_self0, w_neigh0):
    n = B_ * NS1_
    return pl.pallas_call(
        _hh_body,
        in_specs=[
            pl.BlockSpec((n, D_), lambda: (0, 0)),
            pl.BlockSpec((n, D_), lambda: (0, 0)),
            pl.BlockSpec((D_, D_), lambda: (0, 0)),
            pl.BlockSpec((D_, D_), lambda: (0, 0)),
        ],
        out_specs=pl.BlockSpec((n, 2 * D_), lambda: (0, 0)),
        out_shape=jax.ShapeDtypeStruct((n, 2 * D_), jnp.float32),
    )(h1, agg1, w_self0, w_neigh0)


def _attn(self_vecs, neigh, w_att, temp):
    q = jnp.dot(self_vecs, w_att, preferred_element_type=jnp.float32,
                precision=_HP)
    scores = jnp.sum(q[:, None, :] * neigh, axis=-1) / temp
    m = jnp.max(scores, axis=-1, keepdims=True)
    e = jnp.exp(scores - m)
    att = e / jnp.sum(e, axis=-1, keepdims=True)
    return jnp.sum(att[:, :, None] * neigh, axis=1)


def _argmax_rows(x):
    ids = lax.broadcasted_iota(jnp.int32, x.shape, 1)
    m = jnp.max(x, axis=1, keepdims=True)
    return jnp.min(jnp.where(x == m, ids, x.shape[1]), axis=1)


def _head_body(h0_ref, h1_ref, hh1_ref, labels_ref, temp_ref,
               ws0_ref, wn0_ref, wa0_ref, ws1_ref, wn1_ref, wa1_ref,
               wp_ref, bp_ref, preds_ref, loss_ref, acc_ref):
    temp = temp_ref[0]
    h0 = h0_ref[...]
    n0 = h1_ref[...]
    agg0 = _attn(h0, n0, wa0_ref[...], temp)
    fs0 = jnp.dot(h0, ws0_ref[...], preferred_element_type=jnp.float32,
                  precision=_HP)
    fn0 = jnp.dot(agg0, wn0_ref[...], preferred_element_type=jnp.float32,
                  precision=_HP)
    hid0 = jnp.maximum(jnp.concatenate([fs0, fn0], axis=-1), 0.0)
    n1 = hh1_ref[...]
    agg1 = _attn(hid0, n1, wa1_ref[...], temp)
    fs1 = jnp.dot(hid0, ws1_ref[...], preferred_element_type=jnp.float32,
                  precision=_HP)
    fn1 = jnp.dot(agg1, wn1_ref[...], preferred_element_type=jnp.float32,
                  precision=_HP)
    h = jnp.concatenate([fs1, fn1], axis=-1)
    nrm = jnp.sqrt(jnp.sum(h * h, axis=1, keepdims=True)) + 1e-12
    out1 = h / nrm
    logits = jnp.dot(out1, wp_ref[...], preferred_element_type=jnp.float32,
                     precision=_HP) + bp_ref[...]
    lm = jnp.max(logits, axis=-1, keepdims=True)
    ls = logits - lm
    lse = jnp.log(jnp.sum(jnp.exp(ls), axis=-1, keepdims=True))
    logp = ls - lse
    labels = labels_ref[...]
    cross = jnp.mean(-jnp.sum(labels * logp, axis=-1))
    l2 = (jnp.sum(ws0_ref[...] ** 2) + jnp.sum(wn0_ref[...] ** 2)
          + jnp.sum(wa0_ref[...] ** 2) + jnp.sum(ws1_ref[...] ** 2)
          + jnp.sum(wn1_ref[...] ** 2) + jnp.sum(wa1_ref[...] ** 2)
          + jnp.sum(wp_ref[...] ** 2) + jnp.sum(bp_ref[...] ** 2))
    loss = cross + WD_ * 0.5 * l2
    preds = jnp.exp(logp)
    preds_ref[...] = preds
    loss_ref[...] = jnp.reshape(loss, (1, 1))
    acc = jnp.mean((_argmax_rows(preds) == _argmax_rows(labels))
                   .astype(jnp.float32))
    acc_ref[...] = jnp.reshape(acc, (1, 1))


def _head_call(h0, h1r, hh1r, labels, temp, ws0, wn0, wa0, ws1, wn1, wa1,
               wp, bp):
    return pl.pallas_call(
        _head_body,
        in_specs=[
            pl.BlockSpec((B_, D_), lambda: (0, 0)),
            pl.BlockSpec((B_, NS1_, D_), lambda: (0, 0, 0)),
            pl.BlockSpec((B_, NS1_, 2 * D_), lambda: (0, 0, 0)),
            pl.BlockSpec((B_, NCLS_), lambda: (0, 0)),
            pl.BlockSpec(memory_space=pltpu.SMEM),
            pl.BlockSpec((D_, D_), lambda: (0, 0)),
            pl.BlockSpec((D_, D_), lambda: (0, 0)),
            pl.BlockSpec((D_, D_), lambda: (0, 0)),
            pl.BlockSpec((2 * D_, D_), lambda: (0, 0)),
            pl.BlockSpec((2 * D_, D_), lambda: (0, 0)),
            pl.BlockSpec((2 * D_, 2 * D_), lambda: (0, 0)),
            pl.BlockSpec((2 * D_, NCLS_), lambda: (0, 0)),
            pl.BlockSpec((1, NCLS_), lambda: (0, 0)),
        ],
        out_specs=[
            pl.BlockSpec((B_, NCLS_), lambda: (0, 0)),
            pl.BlockSpec((1, 1), lambda: (0, 0)),
            pl.BlockSpec((1, 1), lambda: (0, 0)),
        ],
        out_shape=[
            jax.ShapeDtypeStruct((B_, NCLS_), jnp.float32),
            jax.ShapeDtypeStruct((1, 1), jnp.float32),
            jax.ShapeDtypeStruct((1, 1), jnp.float32),
        ],
    )(h0, h1r, hh1r, labels, temp, ws0, wn0, wa0, ws1, wn1, wa1, wp, bp)


def kernel(features, adj, batch, labels, temperature, W_self0, W_neigh0,
           W_att0, W_self1, W_neigh1, W_att1, W_pred, b_pred):
    adj0, h0 = _gather_hop0(adj, features, batch)
    s1 = adj0[:, :NS1_].reshape(-1)
    adj1, h1 = _gather_hop1(adj, features, s1)
    s2 = adj1[:, :NS0_].reshape(-1)
    temp = temperature.reshape(1)
    q1t = _q_call(h1, W_att0, temp)
    agg1 = _fused_agg(features, s2, q1t)
    hh1 = _hh_call(h1, agg1, W_self0, W_neigh0)
    preds, loss, acc = _head_call(
        h0, h1.reshape(B_, NS1_, D_), hh1.reshape(B_, NS1_, 2 * D_),
        labels, temp, W_self0, W_neigh0, W_att0, W_self1, W_neigh1, W_att1,
        W_pred, b_pred.reshape(1, NCLS_))
    return preds, jnp.reshape(loss, ()), jnp.reshape(acc, ())


# speedup vs baseline: 6.4734x; 1.2802x over previous
import functools

import jax
import jax.numpy as jnp
from jax import lax
from jax.experimental import pallas as pl
from jax.experimental.pallas import tpu as pltpu
from jax.experimental.pallas import tpu_sc as plsc

N_NODES_ = 100000
D_ = 128
MAX_DEG_ = 32
B_ = 1024
NS0_ = 25
NS1_ = 10
NCLS_ = 64
WD_ = 1e-4

_NW = 32
_HP = lax.Precision.DEFAULT


_BPW = B_ // _NW
_S1PW = _BPW * NS1_
_S2PW = _S1PW * NS0_


def _make_sample_gather():
    mesh = plsc.VectorSubcoreMesh(core_axis_name="c", subcore_axis_name="s",
                                  num_cores=2, num_subcores=16)
    s1_streams = ((0, 128), (128, 128), (256, 64))

    @functools.partial(
        pl.kernel, mesh=mesh,
        out_type=[jax.ShapeDtypeStruct((B_, D_), jnp.float32),
                  jax.ShapeDtypeStruct((B_ * NS1_, D_), jnp.float32),
                  jax.ShapeDtypeStruct((B_ * NS1_ * NS0_,), jnp.int32)],
        scratch_types=[
            pltpu.VMEM((_BPW,), jnp.int32),
            pltpu.VMEM((_BPW, MAX_DEG_), jnp.int32),
            pltpu.VMEM((_BPW, D_), jnp.float32),
            pltpu.VMEM((_S1PW,), jnp.int32),
            pltpu.VMEM((_S1PW, MAX_DEG_), jnp.int32),
            pltpu.VMEM((_S1PW, D_), jnp.float32),
            pltpu.VMEM((_S2PW,), jnp.int32),
            pltpu.SemaphoreType.DMA,
            pltpu.SemaphoreType.DMA,
        ],
        compiler_params=pltpu.CompilerParams(use_tc_tiling_on_sc=False,
                                             needs_layout_passes=False))
    def sample(adj_hbm, feat_hbm, batch_hbm, h0_out, h1_out, s2_out,
               bidx_v, adj0_v, h0_v, s1_v, adj1_v, h1_v, s2_v, sem1, sem2):
        wid = lax.axis_index("s") * 2 + lax.axis_index("c")
        iota = lax.iota(jnp.int32, 16)
        b0 = wid * _BPW
        pltpu.sync_copy(batch_hbm.at[pl.ds(b0, _BPW)], bidx_v)
        cp1 = pltpu.async_copy(adj_hbm.at[bidx_v], adj0_v, sem1)
        cp2 = pltpu.async_copy(feat_hbm.at[bidx_v], h0_v, sem2)
        cp1.wait()
        cp2.wait()
        pltpu.sync_copy(h0_v, h0_out.at[pl.ds(b0, _BPW)])
        for c in range(_S1PW // 16):
            iv = iota + c * 16
            r = iv // NS1_
            col = iv - r * NS1_
            s1_v[pl.ds(c * 16, 16)] = plsc.load_gather(adj0_v, [r, col])
        for o, ln in s1_streams:
            pltpu.async_copy(adj_hbm.at[s1_v.at[pl.ds(o, ln)]],
                             adj1_v.at[pl.ds(o, ln)], sem1)
            pltpu.async_copy(feat_hbm.at[s1_v.at[pl.ds(o, ln)]],
                             h1_v.at[pl.ds(o, ln)], sem2)
        for o, ln in s1_streams:
            pltpu.make_async_copy(adj_hbm.at[s1_v.at[pl.ds(o, ln)]],
                                  adj1_v.at[pl.ds(o, ln)], sem1).wait()
            pltpu.make_async_copy(feat_hbm.at[s1_v.at[pl.ds(o, ln)]],
                                  h1_v.at[pl.ds(o, ln)], sem2).wait()
        pltpu.sync_copy(h1_v, h1_out.at[pl.ds(wid * _S1PW, _S1PW)])

        def s2_body(c, carry):
            iv = iota + c * 16
            r = iv // NS0_
            col = iv - r * NS0_
            s2_v[pl.ds(c * 16, 16)] = plsc.load_gather(adj1_v, [r, col])
            return carry

        lax.fori_loop(0, _S2PW // 16, s2_body, 0, unroll=4)
        pltpu.sync_copy(s2_v, s2_out.at[pl.ds(wid * _S2PW, _S2PW)])

    return sample


_sample_gather = _make_sample_gather()


_NT = B_ * NS1_
_TPW = _NT // _NW
_CT = 8
_CI = _CT * NS0_
_NCH = _TPW // _CT


def _make_fused_agg():
    mesh = plsc.VectorSubcoreMesh(core_axis_name="c", subcore_axis_name="s", num_cores=2, num_subcores=16)

    @functools.partial(
        pl.kernel, mesh=mesh,
        out_type=jax.ShapeDtypeStruct((_NT, D_), jnp.float32),
        scratch_types=[
            pltpu.VMEM((_TPW * NS0_,), jnp.int32),
            pltpu.VMEM((_TPW, D_), jnp.float32),
            pltpu.VMEM((2, _CI, D_), jnp.float32),
            pltpu.VMEM((32,), jnp.float32),
            pltpu.VMEM((2, _CT, D_), jnp.float32),
            pltpu.SemaphoreType.DMA,
            pltpu.SemaphoreType.DMA,
            pltpu.SemaphoreType.DMA,
            pltpu.SemaphoreType.DMA,
        ],
        compiler_params=pltpu.CompilerParams(use_tc_tiling_on_sc=False,
                                             needs_layout_passes=False))
    def fused(feat_hbm, s2_hbm, q_hbm, out_hbm,
              idx_a, q_a, rows_v, sc_v, out_v, sg0, sg1, so0, so1):
        wid = lax.axis_index("s") * 2 + lax.axis_index("c")
        base = wid * _TPW
        lanes = lax.iota(jnp.int32, 16)
        sg = (sg0, sg1)
        so = (so0, so1)

        pltpu.sync_copy(s2_hbm.at[pl.ds(base * NS0_, _TPW * NS0_)], idx_a)
        pltpu.sync_copy(q_hbm.at[pl.ds(base, _TPW)], q_a)

        def fire(c, b):
            o = c * _CI
            pltpu.async_copy(feat_hbm.at[idx_a.at[pl.ds(o, 104)]],
                             rows_v.at[b, pl.ds(0, 104)], sg[b])
            pltpu.async_copy(feat_hbm.at[idx_a.at[pl.ds(o + 104, 96)]],
                             rows_v.at[b, pl.ds(104, 96)], sg[b])

        def drain_gather(c, b):
            o = c * _CI
            pltpu.make_async_copy(feat_hbm.at[idx_a.at[pl.ds(o, 104)]],
                                  rows_v.at[b, pl.ds(0, 104)], sg[b]).wait()
            pltpu.make_async_copy(feat_hbm.at[idx_a.at[pl.ds(o + 104, 96)]],
                                  rows_v.at[b, pl.ds(104, 96)], sg[b]).wait()

        def out_start(c, b):
            pltpu.async_copy(out_v.at[b], out_hbm.at[pl.ds(base + c * _CT,
                                                           _CT)], so[b])

        def out_drain(c, b):
            pltpu.make_async_copy(out_v.at[b],
                                  out_hbm.at[pl.ds(base + c * _CT, _CT)],
                                  so[b]).wait()

        def compute(c, b):
            def target_body(j, carry2):
                rbase = j * NS0_
                tl = c * _CT + j
                qv = [q_a[tl, pl.ds(k * 16, 16)] for k in range(8)]

                def dot_body(k, carry3):
                    v0, v1 = carry3
                    r = rbase + k
                    acc = rows_v[b, r, pl.ds(0, 16)] * qv[0]
                    for m_ in range(1, 8):
                        acc = acc + rows_v[b, r, pl.ds(m_ * 16, 16)] * qv[m_]
                    s = jnp.sum(acc, axis=0)
                    v0 = jnp.where(lanes == k, s, v0)
                    v1 = jnp.where(lanes == k - 16, s, v1)
                    return (v0, v1)

                neg = jnp.full((16,), -3e38, jnp.float32)
                s0, s1 = lax.fori_loop(0, NS0_, dot_body, (neg, neg),
                                       unroll=5)
                m = jnp.max(jnp.maximum(s0, s1), axis=0)
                e0 = jnp.exp(s0 - m)
                e1 = jnp.exp(s1 - m)
                tot = jnp.sum(e0 + e1, axis=0)
                sc_v[pl.ds(0, 16)] = e0 / tot
                sc_v[pl.ds(16, 16)] = e1 / tot

                def wsum_body(k, accs):
                    r = rbase + k
                    w = plsc.load_gather(
                        sc_v, [jnp.full((16,), k, jnp.int32)])
                    return tuple(
                        accs[m_] + rows_v[b, r, pl.ds(m_ * 16, 16)] * w
                        for m_ in range(8))

                zero = jnp.zeros((16,), jnp.float32)
                accs = lax.fori_loop(0, NS0_, wsum_body, (zero,) * 8,
                                     unroll=5)
                for m_ in range(8):
                    out_v[b, j, pl.ds(m_ * 16, 16)] = accs[m_]
                return carry2

            lax.fori_loop(0, _CT, target_body, 0)

        fire(0, 0)

        def loop_body(i, carry):
            c0 = 2 * i
            c1 = 2 * i + 1
            fire(c1, 1)
            drain_gather(c0, 0)

            @pl.when(i >= 1)
            def _():
                out_drain(c0 - 2, 0)

            compute(c0, 0)
            out_start(c0, 0)

            @pl.when(i < (_NCH // 2) - 1)
            def _():
                fire(c1 + 1, 0)

            drain_gather(c1, 1)

            @pl.when(i >= 1)
            def _():
                out_drain(c1 - 2, 1)

            compute(c1, 1)
            out_start(c1, 1)
            return carry

        lax.fori_loop(0, _NCH // 2, loop_body, 0)
        out_drain(_NCH - 2, 0)
        out_drain(_NCH - 1, 1)

    return fused


_fused_agg = _make_fused_agg()


def _q_body(h1_ref, wa_ref, temp_ref, out_ref):
    q = jnp.dot(h1_ref[...], wa_ref[...], preferred_element_type=jnp.float32,
                precision=_HP)
    out_ref[...] = q / temp_ref[0]


def _q_call(h1, w_att0, temp):
    n = B_ * NS1_
    return pl.pallas_call(
        _q_body,
        in_specs=[
            pl.BlockSpec((n, D_), lambda: (0, 0)),
            pl.BlockSpec((D_, D_), lambda: (0, 0)),
            pl.BlockSpec(memory_space=pltpu.SMEM),
        ],
        out_specs=pl.BlockSpec((n, D_), lambda: (0, 0)),
        out_shape=jax.ShapeDtypeStruct((n, D_), jnp.float32),
    )(h1, w_att0, temp)


def _attn(self_vecs, neigh, w_att, temp):
    q = jnp.dot(self_vecs, w_att, preferred_element_type=jnp.float32,
                precision=_HP)
    scores = jnp.sum(q[:, None, :] * neigh, axis=-1) / temp
    m = jnp.max(scores, axis=-1, keepdims=True)
    e = jnp.exp(scores - m)
    att = e / jnp.sum(e, axis=-1, keepdims=True)
    return jnp.sum(att[:, :, None] * neigh, axis=1)


def _argmax_rows(x):
    ids = lax.broadcasted_iota(jnp.int32, x.shape, 1)
    m = jnp.max(x, axis=1, keepdims=True)
    return jnp.min(jnp.where(x == m, ids, x.shape[1]), axis=1)


def _head_body(h0_ref, h1_ref, agg1_ref, labels_ref, temp_ref,
               ws0_ref, wn0_ref, wa0_ref, ws1_ref, wn1_ref, wa1_ref,
               wp_ref, bp_ref, preds_ref, loss_ref, acc_ref):
    temp = temp_ref[0]
    h0 = h0_ref[...]
    h1f = h1_ref[...]
    n0 = h1f.reshape(B_, NS1_, D_)
    agg0 = _attn(h0, n0, wa0_ref[...], temp)
    fs0 = jnp.dot(h0, ws0_ref[...], preferred_element_type=jnp.float32,
                  precision=_HP)
    fn0 = jnp.dot(agg0, wn0_ref[...], preferred_element_type=jnp.float32,
                  precision=_HP)
    hid0 = jnp.maximum(jnp.concatenate([fs0, fn0], axis=-1), 0.0)
    hfs = jnp.dot(h1f, ws0_ref[...], preferred_element_type=jnp.float32,
                  precision=_HP)
    hfn = jnp.dot(agg1_ref[...], wn0_ref[...],
                  preferred_element_type=jnp.float32, precision=_HP)
    hh = jnp.maximum(jnp.concatenate([hfs, hfn], axis=-1), 0.0)
    n1 = hh.reshape(B_, NS1_, 2 * D_)
    aggL1 = _attn(hid0, n1, wa1_ref[...], temp)
    fs1 = jnp.dot(hid0, ws1_ref[...], preferred_element_type=jnp.float32,
                  precision=_HP)
    fn1 = jnp.dot(aggL1, wn1_ref[...], preferred_element_type=jnp.float32,
                  precision=_HP)
    h = jnp.concatenate([fs1, fn1], axis=-1)
    nrm = jnp.sqrt(jnp.sum(h * h, axis=1, keepdims=True)) + 1e-12
    out1 = h / nrm
    logits = jnp.dot(out1, wp_ref[...], preferred_element_type=jnp.float32,
                     precision=_HP) + bp_ref[...]
    lm = jnp.max(logits, axis=-1, keepdims=True)
    ls = logits - lm
    lse = jnp.log(jnp.sum(jnp.exp(ls), axis=-1, keepdims=True))
    logp = ls - lse
    labels = labels_ref[...]
    cross = jnp.mean(-jnp.sum(labels * logp, axis=-1))
    es = jnp.exp(ls)
    preds = es / jnp.sum(es, axis=-1, keepdims=True)
    l2 = (jnp.sum(ws0_ref[...] ** 2) + jnp.sum(wn0_ref[...] ** 2)
          + jnp.sum(wa0_ref[...] ** 2) + jnp.sum(ws1_ref[...] ** 2)
          + jnp.sum(wn1_ref[...] ** 2) + jnp.sum(wa1_ref[...] ** 2)
          + jnp.sum(wp_ref[...] ** 2) + jnp.sum(bp_ref[...] ** 2))
    loss = cross + WD_ * 0.5 * l2
    preds_ref[...] = preds
    loss_ref[...] = jnp.reshape(loss, (1, 1))
    acc = jnp.mean((_argmax_rows(preds) == _argmax_rows(labels))
                   .astype(jnp.float32))
    acc_ref[...] = jnp.reshape(acc, (1, 1))


def _head_call(h0, h1, agg1, labels, temp, ws0, wn0, wa0, ws1, wn1, wa1,
               wp, bp):
    return pl.pallas_call(
        _head_body,
        in_specs=[
            pl.BlockSpec((B_, D_), lambda: (0, 0)),
            pl.BlockSpec((B_ * NS1_, D_), lambda: (0, 0)),
            pl.BlockSpec((B_ * NS1_, D_), lambda: (0, 0)),
            pl.BlockSpec((B_, NCLS_), lambda: (0, 0)),
            pl.BlockSpec(memory_space=pltpu.SMEM),
            pl.BlockSpec((D_, D_), lambda: (0, 0)),
            pl.BlockSpec((D_, D_), lambda: (0, 0)),
            pl.BlockSpec((D_, D_), lambda: (0, 0)),
            pl.BlockSpec((2 * D_, D_), lambda: (0, 0)),
            pl.BlockSpec((2 * D_, D_), lambda: (0, 0)),
            pl.BlockSpec((2 * D_, 2 * D_), lambda: (0, 0)),
            pl.BlockSpec((2 * D_, NCLS_), lambda: (0, 0)),
            pl.BlockSpec((1, NCLS_), lambda: (0, 0)),
        ],
        out_specs=[
            pl.BlockSpec((B_, NCLS_), lambda: (0, 0)),
            pl.BlockSpec((1, 1), lambda: (0, 0)),
            pl.BlockSpec((1, 1), lambda: (0, 0)),
        ],
        out_shape=[
            jax.ShapeDtypeStruct((B_, NCLS_), jnp.float32),
            jax.ShapeDtypeStruct((1, 1), jnp.float32),
            jax.ShapeDtypeStruct((1, 1), jnp.float32),
        ],
    )(h0, h1, agg1, labels, temp, ws0, wn0, wa0, ws1, wn1, wa1, wp, bp)


def kernel(features, adj, batch, labels, temperature, W_self0, W_neigh0,
           W_att0, W_self1, W_neigh1, W_att1, W_pred, b_pred):
    h0, h1, s2 = _sample_gather(adj, features, batch)
    temp = temperature.reshape(1)
    q1t = _q_call(h1, W_att0, temp)
    agg1 = _fused_agg(features, s2, q1t)
    preds, loss, acc = _head_call(
        h0, h1, agg1,
        labels, temp, W_self0, W_neigh0, W_att0, W_self1, W_neigh1, W_att1,
        W_pred, b_pred.reshape(1, NCLS_))
    return preds, jnp.reshape(loss, ()), jnp.reshape(acc, ())
